# baseline (speedup 1.0000x reference)
import jax
import jax.numpy as jnp
from jax import lax
from jax.experimental import pallas as pl
from jax.experimental.pallas import tpu as pltpu

_B = 16384
_C = 1000
_ROWS = 1024
_GRID = _B // _ROWS


def _tc_body(x_ref, o_ref):
    x = x_ref[...]
    cls = lax.broadcasted_iota(jnp.int32, (_ROWS, _C), 1)
    o_ref[...] = (x == cls).astype(jnp.float32)


_onehot_tc = pl.pallas_call(
    _tc_body,
    grid=(_GRID,),
    in_specs=[pl.BlockSpec((_ROWS, 1), lambda i: (i, 0))],
    out_specs=pl.BlockSpec((_ROWS, _C), lambda i: (i, 0)),
    out_shape=jax.ShapeDtypeStruct((_B, _C), jnp.float32),
)


@jax.jit
def kernel(x):
    x = jnp.squeeze(x).astype(jnp.int32).reshape(_B, 1)
    return _onehot_tc(x)
